# baseline (speedup 1.0000x reference)
import functools

import jax
import jax.numpy as jnp
from jax import lax
from jax.experimental import pallas as pl
from jax.experimental.pallas import tpu as pltpu
from jax.experimental.pallas import tpu_sc as plsc

NUM_POS = 28 * 28
EMB = 768
XD = 1536
BATCH = 64

NW = 32
NB_SC = 32
B_PER_W = NB_SC // NW
CHUNK = 8
NCHUNK = NUM_POS // CHUNK
LANES = 16
NVEC = EMB // LANES
NBUF = 4
NSTEP = NCHUNK // NBUF

PBLK = 112


def _sc_body(x_hbm, table_hbm, out_hbm, *refs):
    xbufs = refs[0:NBUF]
    tbufs = refs[NBUF:2 * NBUF]
    lsems = refs[2 * NBUF:3 * NBUF]
    ssems = refs[3 * NBUF:4 * NBUF]
    wid = lax.axis_index("s") * 2 + lax.axis_index("c")
    b0 = wid * B_PER_W

    def load_descs(s, c):
        p0 = c * CHUNK
        descs = [
            pltpu.make_async_copy(
                x_hbm.at[b0 + nb, pl.ds(p0, CHUNK)],
                xbufs[s].at[pl.ds(nb * CHUNK, CHUNK)], lsems[s])
            for nb in range(B_PER_W)
        ]
        descs.append(
            pltpu.make_async_copy(table_hbm.at[pl.ds(p0, CHUNK)],
                                  tbufs[s], lsems[s]))
        return descs

    def store_descs(s, c):
        p0 = c * CHUNK
        return [
            pltpu.make_async_copy(
                xbufs[s].at[pl.ds(nb * CHUNK, CHUNK)],
                out_hbm.at[b0 + nb, pl.ds(p0, CHUNK)], ssems[s])
            for nb in range(B_PER_W)
        ]

    def start(descs):
        for d in descs:
            d.start()

    def wait(descs):
        for d in descs:
            d.wait()

    def compute(s):
        xb, tb = xbufs[s], tbufs[s]

        def row(r, _):
            for j in range(NVEC):
                sl = pl.ds(j * LANES, LANES)
                sx = pl.ds(EMB + j * LANES, LANES)
                tv = tb[r, sl]
                for nb in range(B_PER_W):
                    xb[nb * CHUNK + r, sx] = xb[nb * CHUNK + r, sx] + tv
            return 0

        lax.fori_loop(0, CHUNK, row, 0)

    start(load_descs(0, 0))
    start(load_descs(1, 1))

    def step(i, _):
        for s in range(NBUF):
            c = NBUF * i + s
            wait(load_descs(s, c))
            compute(s)
            start(store_descs(s, c))
            s2 = (s + 2) % NBUF
            if s < 2:
                @pl.when(i >= 1)
                def _():
                    wait(store_descs(s2, c - 2))
            else:
                wait(store_descs(s2, c - 2))
            start(load_descs(s2, c + 2))
        return 0

    lax.fori_loop(0, NSTEP, step, 0)

    last = NSTEP * NBUF
    for s in range(2):
        c = last + s
        wait(load_descs(s, c))
        compute(s)
        start(store_descs(s, c))
    wait(store_descs(2, last - 2))
    wait(store_descs(3, last - 1))
    wait(store_descs(0, last))
    wait(store_descs(1, last + 1))


def _sc_add(x, table):
    mesh = plsc.VectorSubcoreMesh(core_axis_name="c", subcore_axis_name="s")
    f = functools.partial(
        pl.kernel,
        mesh=mesh,
        out_type=jax.ShapeDtypeStruct((NB_SC, NUM_POS, XD), jnp.float32),
        scratch_types=(
            [pltpu.VMEM((B_PER_W * CHUNK, XD), jnp.float32)
             for _ in range(NBUF)]
            + [pltpu.VMEM((CHUNK, EMB), jnp.float32) for _ in range(NBUF)]
            + [pltpu.SemaphoreType.DMA for _ in range(2 * NBUF)]),
    )(_sc_body)
    return f(x, table)


def _tc_body(x_ref, t_ref, o_ref):
    xb = x_ref[0]
    o_ref[0, :, :EMB] = xb[:, :EMB]
    o_ref[0, :, EMB:] = xb[:, EMB:] + t_ref[...]


def _tc_add(x, table):
    return pl.pallas_call(
        _tc_body,
        grid=(NUM_POS // PBLK, BATCH - NB_SC),
        in_specs=[
            pl.BlockSpec((1, PBLK, XD), lambda p, b: (b + NB_SC, p, 0)),
            pl.BlockSpec((PBLK, EMB), lambda p, b: (p, 0)),
        ],
        out_specs=pl.BlockSpec((1, PBLK, XD), lambda p, b: (b, p, 0)),
        out_shape=jax.ShapeDtypeStruct((BATCH - NB_SC, NUM_POS, XD),
                                       jnp.float32),
    )(x, table)


@jax.jit
def _run(x, table):
    return jnp.concatenate([_sc_add(x, table), _tc_add(x, table)], axis=0)


def kernel(x, table):
    return _run(x, table)


# speedup vs baseline: 1.7111x; 1.7111x over previous
import functools

import jax
import jax.numpy as jnp
from jax import lax
from jax.experimental import pallas as pl
from jax.experimental.pallas import tpu as pltpu
from jax.experimental.pallas import tpu_sc as plsc

NUM_POS = 28 * 28
EMB = 768
XD = 1536
BATCH = 64

NW = 32
B_PER_W = BATCH // NW
CHUNK = 4
NCHUNK = NUM_POS // CHUNK
LANES = 16
NVEC = EMB // LANES
NBUF = 8
LOOK = NBUF // 2
NSTEP = NCHUNK // NBUF
REM = NCHUNK - NBUF * NSTEP


def _sc_body(x_hbm, table_hbm, out_hbm, *refs):
    xbufs = refs[0:NBUF]
    tbufs = refs[NBUF:2 * NBUF]
    lsems = refs[2 * NBUF:3 * NBUF]
    ssems = refs[3 * NBUF:4 * NBUF]
    wid = lax.axis_index("s") * 2 + lax.axis_index("c")
    b0 = wid * B_PER_W

    def load_descs(s, c):
        p0 = c * CHUNK
        descs = [
            pltpu.make_async_copy(
                x_hbm.at[b0 + nb, pl.ds(p0, CHUNK)],
                xbufs[s].at[pl.ds(nb * CHUNK, CHUNK)], lsems[s])
            for nb in range(B_PER_W)
        ]
        descs.append(
            pltpu.make_async_copy(table_hbm.at[pl.ds(p0, CHUNK)],
                                  tbufs[s], lsems[s]))
        return descs

    def store_descs(s, c):
        p0 = c * CHUNK
        return [
            pltpu.make_async_copy(
                xbufs[s].at[pl.ds(nb * CHUNK, CHUNK)],
                out_hbm.at[b0 + nb, pl.ds(p0, CHUNK)], ssems[s])
            for nb in range(B_PER_W)
        ]

    def start(descs):
        for d in descs:
            d.start()

    def wait(descs):
        for d in descs:
            d.wait()

    def compute(s):
        xb, tb = xbufs[s], tbufs[s]

        def row(r, _):
            for j in range(NVEC):
                sl = pl.ds(j * LANES, LANES)
                sx = pl.ds(EMB + j * LANES, LANES)
                tv = tb[r, sl]
                for nb in range(B_PER_W):
                    xb[nb * CHUNK + r, sx] = xb[nb * CHUNK + r, sx] + tv
            return 0

        lax.fori_loop(0, CHUNK, row, 0)

    for s in range(LOOK):
        start(load_descs(s, s))

    def step(i, _):
        for s in range(NBUF):
            c = NBUF * i + s
            wait(load_descs(s, c))
            compute(s)
            start(store_descs(s, c))
            s2 = (s + LOOK) % NBUF
            if s < NBUF - LOOK:
                @pl.when(i >= 1)
                def _():
                    wait(store_descs(s2, c + LOOK - NBUF))
            else:
                wait(store_descs(s2, c + LOOK - NBUF))

            @pl.when(c + LOOK < NCHUNK)
            def _():
                start(load_descs(s2, c + LOOK))
        return 0

    lax.fori_loop(0, NSTEP, step, 0)

    last = NSTEP * NBUF
    for k in range(REM):
        c = last + k
        wait(load_descs(c % NBUF, c))
        compute(c % NBUF)
        start(store_descs(c % NBUF, c))
    for c in range(last - (NBUF - LOOK), NCHUNK):
        wait(store_descs(c % NBUF, c))


def _sc_add(x, table):
    mesh = plsc.VectorSubcoreMesh(core_axis_name="c", subcore_axis_name="s")
    f = functools.partial(
        pl.kernel,
        mesh=mesh,
        out_type=jax.ShapeDtypeStruct((BATCH, NUM_POS, XD), jnp.float32),
        scratch_types=(
            [pltpu.VMEM((B_PER_W * CHUNK, XD), jnp.float32)
             for _ in range(NBUF)]
            + [pltpu.VMEM((CHUNK, EMB), jnp.float32) for _ in range(NBUF)]
            + [pltpu.SemaphoreType.DMA for _ in range(2 * NBUF)]),
    )(_sc_body)
    return f(x, table)


@jax.jit
def _run(x, table):
    return _sc_add(x, table)


def kernel(x, table):
    return _run(x, table)


# speedup vs baseline: 1.7798x; 1.0402x over previous
import functools

import jax
import jax.numpy as jnp
from jax import lax
from jax.experimental import pallas as pl
from jax.experimental.pallas import tpu as pltpu
from jax.experimental.pallas import tpu_sc as plsc

NUM_POS = 28 * 28
EMB = 768
XD = 1536
BATCH = 64

HALF_POS = NUM_POS // 2
B_PER_W = 4
CHUNK = 8
NCH = HALF_POS // CHUNK
NBUF = 4
LOOK = 2
NITEM = NCH * B_PER_W
LANES = 16
NVEC = EMB // LANES
NROUND = (NCH - 1) // 2


def _sc_body(x_hbm, table_hbm, out_hbm, *refs):
    xbufs = refs[0:NBUF]
    tbufs = refs[NBUF:NBUF + 2]
    lsems = refs[NBUF + 2:2 * NBUF + 2]
    ssems = refs[2 * NBUF + 2:3 * NBUF + 2]
    tsems = refs[3 * NBUF + 2:3 * NBUF + 4]
    core = lax.axis_index("c")
    sub = lax.axis_index("s")
    pbase = core * HALF_POS
    b0 = sub * B_PER_W

    def item_ci_nb(it):
        return it // B_PER_W, it % B_PER_W

    def load_desc(slot, it):
        ci, nb = item_ci_nb(it)
        p0 = pbase + ci * CHUNK
        return (pltpu.make_async_copy(
            x_hbm.at[b0 + nb, pl.ds(p0, CHUNK)],
            xbufs[slot], lsems[slot]),)

    def store_desc(slot, it):
        ci, nb = item_ci_nb(it)
        p0 = pbase + ci * CHUNK
        return (pltpu.make_async_copy(
            xbufs[slot],
            out_hbm.at[b0 + nb, pl.ds(p0, CHUNK)], ssems[slot]),)

    def tload_desc(ci, tslot):
        p0 = pbase + ci * CHUNK
        return (pltpu.make_async_copy(table_hbm.at[pl.ds(p0, CHUNK)],
                                      tbufs[tslot], tsems[tslot]),)

    def start(descs):
        for d in descs:
            d.start()

    def wait(descs):
        for d in descs:
            d.wait()

    def compute(slot, tslot):
        xb, tb = xbufs[slot], tbufs[tslot]

        def row(r, _):
            for j in range(NVEC):
                sl = pl.ds(j * LANES, LANES)
                sx = pl.ds(EMB + j * LANES, LANES)
                xb[r, sx] = xb[r, sx] + tb[r, sl]
            return 0

        lax.fori_loop(0, CHUNK, row, 0)

    def item_step(it, i, k, nb, epilogue):
        if nb == 0:
            wait(tload_desc(0, k))
        wait(load_desc(nb, it))

        s2 = (nb + LOOK) % NBUF
        if not epilogue and k == 0 and nb < 2:
            @pl.when(i >= 1)
            def _():
                wait(store_desc(s2, it - LOOK))
        else:
            wait(store_desc(s2, it - LOOK))

        compute(nb, k)
        start(store_desc(nb, it))
        if (not epilogue) or (it + LOOK < NITEM):
            start(load_desc(s2, it + LOOK))

    start(tload_desc(0, 0))
    for it in range(LOOK):
        start(load_desc(it, it))

    def round_step(i, _):
        for k in (0, 1):
            ci = 2 * i + k
            for nb in range(B_PER_W):
                it = NBUF * ci + nb
                if nb == 0:
                    @pl.when(ci + 1 < NCH)
                    def _():
                        start(tload_desc(ci + 1, 1 - k))
                item_step(it, i, k, nb, epilogue=False)
        return 0

    lax.fori_loop(0, NROUND, round_step, 0)

    ci = NCH - 1
    for nb in range(B_PER_W):
        it = NBUF * ci + nb
        item_step(it, NROUND, 0, nb, epilogue=True)
    wait(store_desc((NITEM - 2) % NBUF, NITEM - 2))
    wait(store_desc((NITEM - 1) % NBUF, NITEM - 1))


def _sc_add(x, table):
    mesh = plsc.VectorSubcoreMesh(core_axis_name="c", subcore_axis_name="s")
    f = functools.partial(
        pl.kernel,
        mesh=mesh,
        out_type=jax.ShapeDtypeStruct((BATCH, NUM_POS, XD), jnp.float32),
        scratch_types=(
            [pltpu.VMEM((CHUNK, XD), jnp.float32) for _ in range(NBUF)]
            + [pltpu.VMEM((CHUNK, EMB), jnp.float32) for _ in range(2)]
            + [pltpu.SemaphoreType.DMA for _ in range(2 * NBUF + 2)]),
    )(_sc_body)
    return f(x, table)


@jax.jit
def _run(x, table):
    return _sc_add(x, table)


def kernel(x, table):
    return _run(x, table)


# speedup vs baseline: 1.9739x; 1.1091x over previous
import functools

import jax
import jax.numpy as jnp
from jax import lax
from jax.experimental import pallas as pl
from jax.experimental.pallas import tpu as pltpu
from jax.experimental.pallas import tpu_sc as plsc

NUM_POS = 28 * 28
EMB = 768
XD = 1536
BATCH = 64

HALF_POS = NUM_POS // 2
B_PER_W = 4
CHUNK = 8
NCH = HALF_POS // CHUNK
NBUF = 4
LOOK = 2
NITEM = NCH * B_PER_W
LANES = 16
NVEC = EMB // LANES
NROUND = (NCH - 1) // 2


def _sc_body(x_hbm, table_hbm, out_hbm, *refs):
    xbufs = refs[0:NBUF]
    tbufs = refs[NBUF:NBUF + 2]
    tshared = refs[NBUF + 2]
    lsems = refs[NBUF + 3:2 * NBUF + 3]
    ssems = refs[2 * NBUF + 3:3 * NBUF + 3]
    tsems = refs[3 * NBUF + 3:3 * NBUF + 5]
    core = lax.axis_index("c")
    sub = lax.axis_index("s")
    pbase = core * HALF_POS
    b0 = sub * B_PER_W

    @pl.when(sub == 0)
    def _():
        pltpu.sync_copy(table_hbm.at[pl.ds(pbase, HALF_POS)], tshared)
    plsc.subcore_barrier()

    def item_ci_nb(it):
        return it // B_PER_W, it % B_PER_W

    def load_desc(slot, it):
        ci, nb = item_ci_nb(it)
        p0 = pbase + ci * CHUNK
        return (pltpu.make_async_copy(
            x_hbm.at[b0 + nb, pl.ds(p0, CHUNK)],
            xbufs[slot], lsems[slot]),)

    def store_desc(slot, it):
        ci, nb = item_ci_nb(it)
        p0 = pbase + ci * CHUNK
        return (pltpu.make_async_copy(
            xbufs[slot],
            out_hbm.at[b0 + nb, pl.ds(p0, CHUNK)], ssems[slot]),)

    def tload_desc(ci, tslot):
        return (pltpu.make_async_copy(tshared.at[pl.ds(ci * CHUNK, CHUNK)],
                                      tbufs[tslot], tsems[tslot]),)

    def start(descs):
        for d in descs:
            d.start()

    def wait(descs):
        for d in descs:
            d.wait()

    def compute(slot, tslot):
        xb, tb = xbufs[slot], tbufs[tslot]

        def row(r, _):
            for j in range(NVEC):
                sl = pl.ds(j * LANES, LANES)
                sx = pl.ds(EMB + j * LANES, LANES)
                xb[r, sx] = xb[r, sx] + tb[r, sl]
            return 0

        lax.fori_loop(0, CHUNK, row, 0)

    def item_step(it, i, k, nb, epilogue):
        if nb == 0:
            wait(tload_desc(0, k))
        wait(load_desc(nb, it))

        s2 = (nb + LOOK) % NBUF
        if not epilogue and k == 0 and nb < 2:
            @pl.when(i >= 1)
            def _():
                wait(store_desc(s2, it - LOOK))
        else:
            wait(store_desc(s2, it - LOOK))

        compute(nb, k)
        start(store_desc(nb, it))
        if (not epilogue) or (it + LOOK < NITEM):
            start(load_desc(s2, it + LOOK))

    start(tload_desc(0, 0))
    for it in range(LOOK):
        start(load_desc(it, it))

    def round_step(i, _):
        for k in (0, 1):
            ci = 2 * i + k
            for nb in range(B_PER_W):
                it = NBUF * ci + nb
                if nb == 0:
                    @pl.when(ci + 1 < NCH)
                    def _():
                        start(tload_desc(ci + 1, 1 - k))
                item_step(it, i, k, nb, epilogue=False)
        return 0

    lax.fori_loop(0, NROUND, round_step, 0)

    ci = NCH - 1
    for nb in range(B_PER_W):
        it = NBUF * ci + nb
        item_step(it, NROUND, 0, nb, epilogue=True)
    wait(store_desc((NITEM - 2) % NBUF, NITEM - 2))
    wait(store_desc((NITEM - 1) % NBUF, NITEM - 1))


def _sc_add(x, table):
    mesh = plsc.VectorSubcoreMesh(core_axis_name="c", subcore_axis_name="s")
    f = functools.partial(
        pl.kernel,
        mesh=mesh,
        out_type=jax.ShapeDtypeStruct((BATCH, NUM_POS, XD), jnp.float32),
        scratch_types=(
            [pltpu.VMEM((CHUNK, XD), jnp.float32) for _ in range(NBUF)]
            + [pltpu.VMEM((CHUNK, EMB), jnp.float32) for _ in range(2)]
            + [pltpu.VMEM_SHARED((HALF_POS, EMB), jnp.float32)]
            + [pltpu.SemaphoreType.DMA for _ in range(2 * NBUF + 2)]),
    )(_sc_body)
    return f(x, table)


@jax.jit
def _run(x, table):
    return _sc_add(x, table)


def kernel(x, table):
    return _run(x, table)


# speedup vs baseline: 2.0246x; 1.0256x over previous
import functools

import jax
import jax.numpy as jnp
from jax import lax
from jax.experimental import pallas as pl
from jax.experimental.pallas import tpu as pltpu
from jax.experimental.pallas import tpu_sc as plsc

NUM_POS = 28 * 28
EMB = 768
XD = 1536
BATCH = 64

HALF_POS = NUM_POS // 2
B_PER_W = 4
CHUNK = 8
NCH = HALF_POS // CHUNK
NBUF = 6
LOOK = 3
TSLOTS = 3
NITEM = NCH * B_PER_W
LANES = 16
NVEC = EMB // LANES
NROUND = (NCH - 1) // TSLOTS


def _sc_body(x_hbm, table_hbm, out_hbm, *refs):
    xbufs = refs[0:NBUF]
    tbufs = refs[NBUF:NBUF + TSLOTS]
    tshared = refs[NBUF + TSLOTS]
    lsems = refs[NBUF + TSLOTS + 1:2 * NBUF + TSLOTS + 1]
    ssems = refs[2 * NBUF + TSLOTS + 1:3 * NBUF + TSLOTS + 1]
    tsems = refs[3 * NBUF + TSLOTS + 1:3 * NBUF + 2 * TSLOTS + 1]
    core = lax.axis_index("c")
    sub = lax.axis_index("s")
    pbase = core * HALF_POS
    b0 = sub * B_PER_W

    @pl.when(sub == 0)
    def _():
        pltpu.sync_copy(table_hbm.at[pl.ds(pbase, HALF_POS)], tshared)
    plsc.subcore_barrier()

    def item_ci_nb(it):
        return it // B_PER_W, it % B_PER_W

    def load_desc(slot, it):
        ci, nb = item_ci_nb(it)
        p0 = pbase + ci * CHUNK
        return (pltpu.make_async_copy(
            x_hbm.at[b0 + nb, pl.ds(p0, CHUNK)],
            xbufs[slot], lsems[slot]),)

    def store_desc(slot, it):
        ci, nb = item_ci_nb(it)
        p0 = pbase + ci * CHUNK
        return (pltpu.make_async_copy(
            xbufs[slot],
            out_hbm.at[b0 + nb, pl.ds(p0, CHUNK)], ssems[slot]),)

    def tload_desc(ci, tslot):
        return (pltpu.make_async_copy(tshared.at[pl.ds(ci * CHUNK, CHUNK)],
                                      tbufs[tslot], tsems[tslot]),)

    def start(descs):
        for d in descs:
            d.start()

    def wait(descs):
        for d in descs:
            d.wait()

    def compute(slot, tslot):
        xb, tb = xbufs[slot], tbufs[tslot]

        def row(r, _):
            for j in range(NVEC):
                sl = pl.ds(j * LANES, LANES)
                sx = pl.ds(EMB + j * LANES, LANES)
                xb[r, sx] = xb[r, sx] + tb[r, sl]
            return 0

        lax.fori_loop(0, CHUNK, row, 0)

    def item_step(it, i, o, k, nb, epilogue):
        slot = o % NBUF
        s2 = (slot + LOOK) % NBUF
        if nb == 0:
            wait(tload_desc(0, k))
        wait(load_desc(slot, it))

        if not epilogue and o < LOOK:
            @pl.when(i >= 1)
            def _():
                wait(store_desc(s2, it - LOOK))
        else:
            wait(store_desc(s2, it - LOOK))

        compute(slot, k)
        start(store_desc(slot, it))
        if (not epilogue) or (it + LOOK < NITEM):
            start(load_desc(s2, it + LOOK))

    start(tload_desc(0, 0))
    for it in range(LOOK):
        start(load_desc(it, it))

    def round_step(i, _):
        for k in range(TSLOTS):
            ci = TSLOTS * i + k
            for nb in range(B_PER_W):
                o = B_PER_W * k + nb
                it = B_PER_W * ci + nb
                if nb == 0:
                    start(tload_desc(ci + 1, (k + 1) % TSLOTS))
                item_step(it, i, o, k, nb, epilogue=False)
        return 0

    lax.fori_loop(0, NROUND, round_step, 0)

    ci = NCH - 1
    for nb in range(B_PER_W):
        it = B_PER_W * ci + nb
        item_step(it, NROUND, it % (NBUF * 2), 0, nb, epilogue=True)
    for it in range(NITEM - LOOK, NITEM):
        wait(store_desc(it % NBUF, it))


def _sc_add(x, table):
    mesh = plsc.VectorSubcoreMesh(core_axis_name="c", subcore_axis_name="s")
    f = functools.partial(
        pl.kernel,
        mesh=mesh,
        out_type=jax.ShapeDtypeStruct((BATCH, NUM_POS, XD), jnp.float32),
        scratch_types=(
            [pltpu.VMEM((CHUNK, XD), jnp.float32) for _ in range(NBUF)]
            + [pltpu.VMEM((CHUNK, EMB), jnp.float32) for _ in range(TSLOTS)]
            + [pltpu.VMEM_SHARED((HALF_POS, EMB), jnp.float32)]
            + [pltpu.SemaphoreType.DMA for _ in range(2 * NBUF + TSLOTS)]),
    )(_sc_body)
    return f(x, table)


@jax.jit
def _run(x, table):
    return _sc_add(x, table)


def kernel(x, table):
    return _run(x, table)


# speedup vs baseline: 2.0426x; 1.0089x over previous
import functools

import jax
import jax.numpy as jnp
from jax import lax
from jax.experimental import pallas as pl
from jax.experimental.pallas import tpu as pltpu
from jax.experimental.pallas import tpu_sc as plsc

NUM_POS = 28 * 28
EMB = 768
XD = 1536
BATCH = 64

HALF_POS = NUM_POS // 2
B_PER_W = 4
CHUNK = 8
NCH = HALF_POS // CHUNK
NBUF = 8
LOOK = 4
TSLOTS = 2
NITEM = NCH * B_PER_W
LANES = 16
NVEC = EMB // LANES
NROUND = (NCH - 1) // TSLOTS


def _sc_body(x_hbm, table_hbm, out_hbm, *refs):
    xbufs = refs[0:NBUF]
    tbufs = refs[NBUF:NBUF + TSLOTS]
    tshared = refs[NBUF + TSLOTS]
    lsems = refs[NBUF + TSLOTS + 1:2 * NBUF + TSLOTS + 1]
    ssems = refs[2 * NBUF + TSLOTS + 1:3 * NBUF + TSLOTS + 1]
    tsems = refs[3 * NBUF + TSLOTS + 1:3 * NBUF + 2 * TSLOTS + 1]
    core = lax.axis_index("c")
    sub = lax.axis_index("s")
    pbase = core * HALF_POS
    b0 = sub * B_PER_W

    @pl.when(sub == 0)
    def _():
        pltpu.sync_copy(table_hbm.at[pl.ds(pbase, HALF_POS)], tshared)
    plsc.subcore_barrier()

    def item_ci_nb(it):
        return it // B_PER_W, it % B_PER_W

    def load_desc(slot, it):
        ci, nb = item_ci_nb(it)
        p0 = pbase + ci * CHUNK
        return (pltpu.make_async_copy(
            x_hbm.at[b0 + nb, pl.ds(p0, CHUNK)],
            xbufs[slot], lsems[slot]),)

    def store_desc(slot, it):
        ci, nb = item_ci_nb(it)
        p0 = pbase + ci * CHUNK
        return (pltpu.make_async_copy(
            xbufs[slot],
            out_hbm.at[b0 + nb, pl.ds(p0, CHUNK)], ssems[slot]),)

    def tload_desc(ci, tslot):
        return (pltpu.make_async_copy(tshared.at[pl.ds(ci * CHUNK, CHUNK)],
                                      tbufs[tslot], tsems[tslot]),)

    def start(descs):
        for d in descs:
            d.start()

    def wait(descs):
        for d in descs:
            d.wait()

    def compute(slot, tslot):
        xb, tb = xbufs[slot], tbufs[tslot]

        def row(r, _):
            for j in range(NVEC):
                sl = pl.ds(j * LANES, LANES)
                sx = pl.ds(EMB + j * LANES, LANES)
                xb[r, sx] = xb[r, sx] + tb[r, sl]
            return 0

        lax.fori_loop(0, CHUNK, row, 0)

    def item_step(it, i, o, k, nb, epilogue):
        slot = o % NBUF
        s2 = (slot + LOOK) % NBUF
        if nb == 0:
            wait(tload_desc(0, k))
        wait(load_desc(slot, it))

        if not epilogue and o < LOOK:
            @pl.when(i >= 1)
            def _():
                wait(store_desc(s2, it - LOOK))
        else:
            wait(store_desc(s2, it - LOOK))

        compute(slot, k)
        start(store_desc(slot, it))
        if (not epilogue) or (it + LOOK < NITEM):
            start(load_desc(s2, it + LOOK))

    start(tload_desc(0, 0))
    for it in range(LOOK):
        start(load_desc(it, it))

    def round_step(i, _):
        for k in range(TSLOTS):
            ci = TSLOTS * i + k
            for nb in range(B_PER_W):
                o = B_PER_W * k + nb
                it = B_PER_W * ci + nb
                if nb == 0:
                    start(tload_desc(ci + 1, (k + 1) % TSLOTS))
                item_step(it, i, o, k, nb, epilogue=False)
        return 0

    lax.fori_loop(0, NROUND, round_step, 0)

    ci = NCH - 1
    for nb in range(B_PER_W):
        it = B_PER_W * ci + nb
        item_step(it, NROUND, it % (NBUF * 2), 0, nb, epilogue=True)
    for it in range(NITEM - LOOK, NITEM):
        wait(store_desc(it % NBUF, it))


def _sc_add(x, table):
    mesh = plsc.VectorSubcoreMesh(core_axis_name="c", subcore_axis_name="s")
    f = functools.partial(
        pl.kernel,
        mesh=mesh,
        out_type=jax.ShapeDtypeStruct((BATCH, NUM_POS, XD), jnp.float32),
        scratch_types=(
            [pltpu.VMEM((CHUNK, XD), jnp.float32) for _ in range(NBUF)]
            + [pltpu.VMEM((CHUNK, EMB), jnp.float32) for _ in range(TSLOTS)]
            + [pltpu.VMEM_SHARED((HALF_POS, EMB), jnp.float32)]
            + [pltpu.SemaphoreType.DMA for _ in range(2 * NBUF + TSLOTS)]),
    )(_sc_body)
    return f(x, table)


@jax.jit
def _run(x, table):
    return _sc_add(x, table)


def kernel(x, table):
    return _run(x, table)
